# baseline (speedup 1.0000x reference)
import functools

import jax
import jax.numpy as jnp
from jax import lax
from jax.experimental import pallas as pl
from jax.experimental.pallas import tpu as pltpu
from jax.experimental.pallas import tpu_sc as plsc

TT = 64
VT = 2048
NC, NS = 2, 16
NW = NC * NS
CH = 128


def _compact_body(v, caw_bf_ref, ids_col_ref, ids_row_ref, enc_ref, w_ref,
                  cd_ref, idx_ref, ew_ref):
    t = caw_bf_ref.shape[1]
    h = enc_ref.shape[2]
    eq = (ids_col_ref[0] == ids_row_ref[0]).astype(jnp.bfloat16)
    cd_ref[0] = jnp.dot(caw_bf_ref[0], eq,
                        preferred_element_type=jnp.float32)
    row = (lax.broadcasted_iota(jnp.int32, (t, 1), 0)
           + pl.program_id(0) * t)
    idx_ref[0] = row * v + ids_row_ref[0]
    ew_ref[0] = jnp.sum(enc_ref[0] * w_ref[0][:h][None, :], axis=1,
                        keepdims=True)


def _dense_body(caw_ref, dec_ref, ew_ref, x_ref, w_ref, b_ref,
                out_ref, m_ref, sc_ref, og_ref):
    tt = caw_ref.shape[1]
    h = dec_ref.shape[2]
    v = x_ref.shape[2]

    w = w_ref[0]
    ew = ew_ref[0][:, 0]
    g = (jnp.sum(caw_ref[0] * ew[None, :], axis=1, keepdims=True)
         + jnp.sum(dec_ref[0] * w[h:][None, :], axis=1, keepdims=True)
         + b_ref[0, 0])
    pg = jax.nn.sigmoid(g)

    starts = list(range(0, v, VT))
    widths = [min(VT, v - s0) for s0 in starts]

    m = jnp.full((tt, 1), -jnp.inf, jnp.float32)
    for s0, w_ in zip(starts, widths):
        m = jnp.maximum(m, jnp.max(x_ref[0, :, s0:s0 + w_], axis=1,
                                   keepdims=True))

    z = jnp.zeros((tt, 1), jnp.float32)
    for s0, w_ in zip(starts, widths):
        e = jnp.exp(x_ref[0, :, s0:s0 + w_] - m)
        out_ref[0, :, s0:s0 + w_] = e
        z = z + jnp.sum(e, axis=1, keepdims=True)

    scale = pg / z
    for s0, w_ in zip(starts, widths):
        e = out_ref[0, :, s0:s0 + w_]
        out_ref[0, :, s0:s0 + w_] = jnp.log(scale * e + 1e-12)

    m_ref[0] = m
    sc_ref[0] = scale
    og_ref[0] = 1.0 - pg


def _fix_body(xg_ref, cd_ref, m_ref, sc_ref, og_ref, vals_ref):
    e = jnp.exp(xg_ref[0] - m_ref[0])
    vals_ref[0] = jnp.log(sc_ref[0] * e + og_ref[0] * cd_ref[0] + 1e-12)


LAG = 64


def _gather_body(x_flat, idx_hbm, xg_hbm, idx_v, xg_v, sem):
    wid = lax.axis_index("s") * NC + lax.axis_index("c")
    pltpu.sync_copy(idx_hbm.at[wid], idx_v)
    k = idx_v.shape[0]

    def fire(j, c):
        pltpu.async_copy(x_flat.at[idx_v.at[j]], xg_v.at[j], sem)

        @pl.when(j >= LAG)
        def _drain_one():
            pltpu.make_async_copy(xg_hbm.at[wid].at[j], xg_v.at[j],
                                  sem).wait()

        return c

    lax.fori_loop(0, k, fire, 0)
    pltpu.make_async_copy(xg_hbm.at[wid].at[pl.ds(0, LAG)],
                          xg_v.at[pl.ds(0, LAG)], sem).wait()
    pltpu.sync_copy(xg_v, xg_hbm.at[wid])


def _patch_body(bq, base_hbm, vals_hbm, ids_hbm, out_hbm, ids_v, vals_v,
                rb, sem):
    t_tgt, v = base_hbm.shape[1], base_hbm.shape[2]
    s_src = ids_hbm.shape[1]
    rows = t_tgt // NW
    wid = lax.axis_index("s") * NC + lax.axis_index("c")
    t0 = wid * rows
    pltpu.sync_copy(ids_hbm.at[0], ids_v)
    pltpu.sync_copy(vals_hbm.at[0, pl.ds(t0, rows), :], vals_v)
    nchunks = s_src // 16

    pltpu.async_copy(base_hbm.at[0, t0, :], rb.at[0], sem)

    def row_step(r, c):
        slot = lax.rem(r, 2)
        pltpu.make_async_copy(base_hbm.at[0, t0, :], rb.at[slot],
                              sem).wait()

        @pl.when(r + 1 < rows)
        def _pf():
            pltpu.async_copy(base_hbm.at[0, t0 + r + 1, :],
                             rb.at[1 - slot], sem)

        def patch(j, c2):
            vidx = ids_v[pl.ds(j * 16, 16)]
            vval = vals_v[r, pl.ds(j * 16, 16)]
            plsc.store_scatter(rb.at[slot], [vidx], vval)
            return c2

        lax.fori_loop(0, nchunks, patch, 0)
        pltpu.sync_copy(rb.at[slot], out_hbm.at[bq, t0 + r, :])
        return c

    lax.fori_loop(0, rows, row_step, 0)


def kernel(decoder_hidden_states, cross_attention_weights,
           encoder_hidden_states, vocab_logits, source_ids, vocab_size,
           W_gen, b_gen):
    b, t_tgt, h = decoder_hidden_states.shape
    t_src = encoder_hidden_states.shape[1]
    v = vocab_logits.shape[2]
    n_idx = b * t_tgt * t_src
    k = n_idx // (NW * CH)

    caw = cross_attention_weights
    caw_bf = caw.astype(jnp.bfloat16)
    ids = source_ids.astype(jnp.int32)
    ids_col = ids.reshape(b, t_src, 1)
    ids_row = ids.reshape(b, 1, t_src)
    w2 = W_gen.reshape(1, 2 * h)
    b2 = b_gen.reshape(1, 1)

    cd, fidx, ew = pl.pallas_call(
        functools.partial(_compact_body, v),
        grid=(b,),
        in_specs=[
            pl.BlockSpec((1, t_tgt, t_src), lambda i: (i, 0, 0)),
            pl.BlockSpec((1, t_src, 1), lambda i: (i, 0, 0)),
            pl.BlockSpec((1, 1, t_src), lambda i: (i, 0, 0)),
            pl.BlockSpec((1, t_src, h), lambda i: (i, 0, 0)),
            pl.BlockSpec((1, 2 * h), lambda i: (0, 0)),
        ],
        out_specs=[
            pl.BlockSpec((1, t_tgt, t_src), lambda i: (i, 0, 0)),
            pl.BlockSpec((1, t_tgt, t_src), lambda i: (i, 0, 0)),
            pl.BlockSpec((1, t_src, 1), lambda i: (i, 0, 0)),
        ],
        out_shape=[
            jax.ShapeDtypeStruct((b, t_tgt, t_src), jnp.float32),
            jax.ShapeDtypeStruct((b, t_tgt, t_src), jnp.int32),
            jax.ShapeDtypeStruct((b, t_src, 1), jnp.float32),
        ],
    )(caw_bf, ids_col, ids_row, encoder_hidden_states, w2)

    mesh = plsc.VectorSubcoreMesh(core_axis_name="c", subcore_axis_name="s",
                                  num_cores=NC, num_subcores=NS)
    idx3 = fidx.reshape(NW, k, CH)
    x_flat = vocab_logits.reshape(-1)
    gather_k = pl.kernel(
        _gather_body,
        out_type=jax.ShapeDtypeStruct((NW, k, CH), jnp.float32),
        mesh=mesh,
        scratch_types=[
            pltpu.VMEM((k, CH), jnp.int32),
            pltpu.VMEM((k, CH), jnp.float32),
            pltpu.SemaphoreType.DMA,
        ],
    )
    xg3 = gather_k(x_flat, idx3).reshape(b, t_tgt, t_src)

    nt = t_tgt // TT
    rows = t_tgt // NW
    oref = None
    for bq in range(b):
        base_b, m_b, sc_b, og_b = pl.pallas_call(
            _dense_body,
            grid=(nt,),
            in_specs=[
                pl.BlockSpec((1, TT, t_src), lambda t, i=bq: (i, t, 0)),
                pl.BlockSpec((1, TT, h), lambda t, i=bq: (i, t, 0)),
                pl.BlockSpec((1, t_src, 1), lambda t, i=bq: (i, 0, 0)),
                pl.BlockSpec((1, TT, v), lambda t, i=bq: (i, t, 0)),
                pl.BlockSpec((1, 2 * h), lambda t: (0, 0)),
                pl.BlockSpec((1, 1), lambda t: (0, 0)),
            ],
            out_specs=[
                pl.BlockSpec((1, TT, v), lambda t: (0, t, 0)),
                pl.BlockSpec((1, TT, 1), lambda t: (0, t, 0)),
                pl.BlockSpec((1, TT, 1), lambda t: (0, t, 0)),
                pl.BlockSpec((1, TT, 1), lambda t: (0, t, 0)),
            ],
            out_shape=[
                jax.ShapeDtypeStruct((1, t_tgt, v), jnp.float32),
                jax.ShapeDtypeStruct((1, t_tgt, 1), jnp.float32),
                jax.ShapeDtypeStruct((1, t_tgt, 1), jnp.float32),
                jax.ShapeDtypeStruct((1, t_tgt, 1), jnp.float32),
            ],
        )(caw, decoder_hidden_states, ew, vocab_logits, w2, b2)

        vals_b = pl.pallas_call(
            _fix_body,
            grid=(1,),
            in_specs=[
                pl.BlockSpec((1, t_tgt, t_src), lambda i: (i, 0, 0)),
                pl.BlockSpec((1, t_tgt, t_src), lambda i: (i, 0, 0)),
                pl.BlockSpec((1, t_tgt, 1), lambda i: (i, 0, 0)),
                pl.BlockSpec((1, t_tgt, 1), lambda i: (i, 0, 0)),
                pl.BlockSpec((1, t_tgt, 1), lambda i: (i, 0, 0)),
            ],
            out_specs=pl.BlockSpec((1, t_tgt, t_src), lambda i: (i, 0, 0)),
            out_shape=jax.ShapeDtypeStruct((1, t_tgt, t_src), jnp.float32),
        )(lax.slice_in_dim(xg3, bq, bq + 1, axis=0),
          lax.slice_in_dim(cd, bq, bq + 1, axis=0), m_b, sc_b, og_b)

        ids_b = lax.slice_in_dim(ids, bq, bq + 1, axis=0)
        patch_k = pl.kernel(
            functools.partial(_patch_body, bq),
            out_type=(jax.ShapeDtypeStruct((b, t_tgt, v), jnp.float32)
                      if bq == 0 else ()),
            mesh=mesh,
            compiler_params=pltpu.CompilerParams(
                use_tc_tiling_on_sc=False, needs_layout_passes=False),
            scratch_types=[
                pltpu.VMEM((t_src,), jnp.int32),
                pltpu.VMEM((rows, t_src), jnp.float32),
                pltpu.VMEM((2, v), jnp.float32),
                pltpu.SemaphoreType.DMA,
            ],
        )
        if bq == 0:
            out0 = patch_k(base_b, vals_b, ids_b)
            oref = jax.new_ref(out0)
        else:
            patch_k(base_b, vals_b, ids_b, oref)

    return oref[...]


# speedup vs baseline: 1.0313x; 1.0313x over previous
import functools

import jax
import jax.numpy as jnp
from jax import lax
from jax.experimental import pallas as pl
from jax.experimental.pallas import tpu as pltpu
from jax.experimental.pallas import tpu_sc as plsc

TT = 32
VT = 2048
NC, NS = 2, 16
NW = NC * NS
CH = 128


def _compact_body(caw_bf_ref, ids_col_ref, ids_row_ref, enc_ref, w_ref,
                  cd_ref, ew_ref):
    h = enc_ref.shape[2]
    eq = (ids_col_ref[0] == ids_row_ref[0]).astype(jnp.bfloat16)
    cd_ref[0] = jnp.dot(caw_bf_ref[0], eq,
                        preferred_element_type=jnp.float32)
    ew_ref[0] = jnp.sum(enc_ref[0] * w_ref[0][:h][None, :], axis=1,
                        keepdims=True)


def _dense_body(caw_ref, dec_ref, ew_ref, x_ref, w_ref, b_ref,
                out_ref, m_ref, sc_ref, og_ref):
    tt = caw_ref.shape[1]
    h = dec_ref.shape[2]
    v = x_ref.shape[2]

    w = w_ref[0]
    ew = ew_ref[0][:, 0]
    g = (jnp.sum(caw_ref[0] * ew[None, :], axis=1, keepdims=True)
         + jnp.sum(dec_ref[0] * w[h:][None, :], axis=1, keepdims=True)
         + b_ref[0, 0])
    pg = jax.nn.sigmoid(g)

    starts = list(range(0, v, VT))
    widths = [min(VT, v - s0) for s0 in starts]

    m = jnp.full((tt, 1), -jnp.inf, jnp.float32)
    for s0, w_ in zip(starts, widths):
        m = jnp.maximum(m, jnp.max(x_ref[0, :, s0:s0 + w_], axis=1,
                                   keepdims=True))

    z = jnp.zeros((tt, 1), jnp.float32)
    for s0, w_ in zip(starts, widths):
        e = jnp.exp(x_ref[0, :, s0:s0 + w_] - m)
        out_ref[0, :, s0:s0 + w_] = e
        z = z + jnp.sum(e, axis=1, keepdims=True)

    scale = pg / z
    for s0, w_ in zip(starts, widths):
        e = out_ref[0, :, s0:s0 + w_]
        out_ref[0, :, s0:s0 + w_] = jnp.log(scale * e + 1e-12)

    m_ref[0] = m
    sc_ref[0] = scale
    og_ref[0] = 1.0 - pg


def _fix_body(xg_ref, cd_ref, m_ref, sc_ref, og_ref, vals_ref):
    e = jnp.exp(xg_ref[0] - m_ref[0])
    vals_ref[0] = jnp.log(sc_ref[0] * e + og_ref[0] * cd_ref[0] + 1e-12)


LAG = 64


def _gather_body(x_flat, idx_hbm, xg_hbm, idx_v, xg_v, sem):
    wid = lax.axis_index("s") * NC + lax.axis_index("c")
    pltpu.sync_copy(idx_hbm.at[wid], idx_v)
    k = idx_v.shape[0]

    def fire(j, c):
        pltpu.async_copy(x_flat.at[idx_v.at[j]], xg_v.at[j], sem)

        @pl.when(j >= LAG)
        def _drain_one():
            pltpu.make_async_copy(xg_hbm.at[wid].at[j], xg_v.at[j],
                                  sem).wait()

        return c

    lax.fori_loop(0, k, fire, 0)
    pltpu.make_async_copy(xg_hbm.at[wid].at[pl.ds(0, LAG)],
                          xg_v.at[pl.ds(0, LAG)], sem).wait()
    pltpu.sync_copy(xg_v, xg_hbm.at[wid])


def _patch_body(bq, base_hbm, vals_hbm, ids_hbm, out_hbm, ids_v, vals_v,
                rb, sem_i0, sem_i1, sem_o0, sem_o1):
    t_tgt, v = base_hbm.shape[1], base_hbm.shape[2]
    s_src = ids_hbm.shape[1]
    rows = t_tgt // NW
    pairs = rows // 2
    wid = lax.axis_index("s") * NC + lax.axis_index("c")
    t0 = wid * rows
    pltpu.sync_copy(ids_hbm.at[0], ids_v)
    pltpu.sync_copy(vals_hbm.at[0, pl.ds(t0, rows), :], vals_v)
    nchunks = s_src // 16
    sem_i = (sem_i0, sem_i1)
    sem_o = (sem_o0, sem_o1)

    def patch_row(r, slot):
        def patch(j, c2):
            vidx = ids_v[pl.ds(j * 16, 16)]
            vval = vals_v[r, pl.ds(j * 16, 16)]
            plsc.store_scatter(rb.at[slot], [vidx], vval)
            return c2

        lax.fori_loop(0, nchunks, patch, 0)

    def drain(slot, sem):
        pltpu.make_async_copy(base_hbm.at[0, t0, :], rb.at[slot],
                              sem).wait()

    pltpu.async_copy(base_hbm.at[0, t0, :], rb.at[0], sem_i0)
    pltpu.async_copy(base_hbm.at[0, t0 + 1, :], rb.at[1], sem_i1)
    for rp in range(pairs):
        for sl in range(2):
            r = 2 * rp + sl
            drain(sl, sem_i[sl])
            patch_row(r, sl)
            pltpu.async_copy(rb.at[sl], out_hbm.at[bq, t0 + r, :],
                             sem_o[sl])
        if rp + 1 < pairs:
            for sl in range(2):
                drain(sl, sem_o[sl])
                pltpu.async_copy(base_hbm.at[0, t0 + 2 * rp + 2 + sl, :],
                                 rb.at[sl], sem_i[sl])
    drain(0, sem_o0)
    drain(1, sem_o1)


def kernel(decoder_hidden_states, cross_attention_weights,
           encoder_hidden_states, vocab_logits, source_ids, vocab_size,
           W_gen, b_gen):
    b, t_tgt, h = decoder_hidden_states.shape
    t_src = encoder_hidden_states.shape[1]
    v = vocab_logits.shape[2]
    n_idx = b * t_tgt * t_src
    k = n_idx // (NW * CH)

    caw = cross_attention_weights
    caw_bf = caw.astype(jnp.bfloat16)
    ids = source_ids.astype(jnp.int32)
    ids_col = ids.reshape(b, t_src, 1)
    ids_row = ids.reshape(b, 1, t_src)
    w2 = W_gen.reshape(1, 2 * h)
    b2 = b_gen.reshape(1, 1)

    cd, ew = pl.pallas_call(
        _compact_body,
        grid=(b,),
        in_specs=[
            pl.BlockSpec((1, t_tgt, t_src), lambda i: (i, 0, 0)),
            pl.BlockSpec((1, t_src, 1), lambda i: (i, 0, 0)),
            pl.BlockSpec((1, 1, t_src), lambda i: (i, 0, 0)),
            pl.BlockSpec((1, t_src, h), lambda i: (i, 0, 0)),
            pl.BlockSpec((1, 2 * h), lambda i: (0, 0)),
        ],
        out_specs=[
            pl.BlockSpec((1, t_tgt, t_src), lambda i: (i, 0, 0)),
            pl.BlockSpec((1, t_src, 1), lambda i: (i, 0, 0)),
        ],
        out_shape=[
            jax.ShapeDtypeStruct((b, t_tgt, t_src), jnp.float32),
            jax.ShapeDtypeStruct((b, t_src, 1), jnp.float32),
        ],
    )(caw_bf, ids_col, ids_row, encoder_hidden_states, w2)

    fidx = ((jnp.arange(b * t_tgt, dtype=jnp.int32) * v)
            .reshape(b, t_tgt, 1) + ids[:, None, :])

    mesh = plsc.VectorSubcoreMesh(core_axis_name="c", subcore_axis_name="s",
                                  num_cores=NC, num_subcores=NS)
    idx3 = fidx.reshape(NW, k, CH)
    x_flat = vocab_logits.reshape(-1)
    gather_k = pl.kernel(
        _gather_body,
        out_type=jax.ShapeDtypeStruct((NW, k, CH), jnp.float32),
        mesh=mesh,
        scratch_types=[
            pltpu.VMEM((k, CH), jnp.int32),
            pltpu.VMEM((k, CH), jnp.float32),
            pltpu.SemaphoreType.DMA,
        ],
    )
    xg3 = gather_k(x_flat, idx3).reshape(b, t_tgt, t_src)

    nt = t_tgt // TT
    rows = t_tgt // NW
    oref = None
    for bq in range(b):
        base_b, m_b, sc_b, og_b = pl.pallas_call(
            _dense_body,
            grid=(nt,),
            in_specs=[
                pl.BlockSpec((1, TT, t_src), lambda t, i=bq: (i, t, 0)),
                pl.BlockSpec((1, TT, h), lambda t, i=bq: (i, t, 0)),
                pl.BlockSpec((1, t_src, 1), lambda t, i=bq: (i, 0, 0)),
                pl.BlockSpec((1, TT, v), lambda t, i=bq: (i, t, 0)),
                pl.BlockSpec((1, 2 * h), lambda t: (0, 0)),
                pl.BlockSpec((1, 1), lambda t: (0, 0)),
            ],
            out_specs=[
                pl.BlockSpec((1, TT, v), lambda t: (0, t, 0)),
                pl.BlockSpec((1, TT, 1), lambda t: (0, t, 0)),
                pl.BlockSpec((1, TT, 1), lambda t: (0, t, 0)),
                pl.BlockSpec((1, TT, 1), lambda t: (0, t, 0)),
            ],
            out_shape=[
                jax.ShapeDtypeStruct((1, t_tgt, v), jnp.float32),
                jax.ShapeDtypeStruct((1, t_tgt, 1), jnp.float32),
                jax.ShapeDtypeStruct((1, t_tgt, 1), jnp.float32),
                jax.ShapeDtypeStruct((1, t_tgt, 1), jnp.float32),
            ],
        )(caw, decoder_hidden_states, ew, vocab_logits, w2, b2)

        vals_b = pl.pallas_call(
            _fix_body,
            grid=(1,),
            in_specs=[
                pl.BlockSpec((1, t_tgt, t_src), lambda i: (i, 0, 0)),
                pl.BlockSpec((1, t_tgt, t_src), lambda i: (i, 0, 0)),
                pl.BlockSpec((1, t_tgt, 1), lambda i: (i, 0, 0)),
                pl.BlockSpec((1, t_tgt, 1), lambda i: (i, 0, 0)),
                pl.BlockSpec((1, t_tgt, 1), lambda i: (i, 0, 0)),
            ],
            out_specs=pl.BlockSpec((1, t_tgt, t_src), lambda i: (i, 0, 0)),
            out_shape=jax.ShapeDtypeStruct((1, t_tgt, t_src), jnp.float32),
        )(lax.slice_in_dim(xg3, bq, bq + 1, axis=0),
          lax.slice_in_dim(cd, bq, bq + 1, axis=0), m_b, sc_b, og_b)

        ids_b = lax.slice_in_dim(ids, bq, bq + 1, axis=0)
        patch_k = pl.kernel(
            functools.partial(_patch_body, bq),
            out_type=(jax.ShapeDtypeStruct((b, t_tgt, v), jnp.float32)
                      if bq == 0 else ()),
            mesh=mesh,
            compiler_params=pltpu.CompilerParams(
                use_tc_tiling_on_sc=False, needs_layout_passes=False),
            scratch_types=[
                pltpu.VMEM((t_src,), jnp.int32),
                pltpu.VMEM((rows, t_src), jnp.float32),
                pltpu.VMEM((2, v), jnp.float32),
                pltpu.SemaphoreType.DMA,
                pltpu.SemaphoreType.DMA,
                pltpu.SemaphoreType.DMA,
                pltpu.SemaphoreType.DMA,
            ],
        )
        if bq == 0:
            out0 = patch_k(base_b, vals_b, ids_b)
            oref = jax.new_ref(out0)
        else:
            patch_k(base_b, vals_b, ids_b, oref)

    return oref[...]


# speedup vs baseline: 1.0676x; 1.0353x over previous
import functools

import jax
import jax.numpy as jnp
from jax import lax
from jax.experimental import pallas as pl
from jax.experimental.pallas import tpu as pltpu
from jax.experimental.pallas import tpu_sc as plsc

TT = 32
VT = 2048
NC, NS = 2, 16
NW = NC * NS
CH = 128


def _compact_body(caw_bf_ref, ids_col_ref, ids_row_ref, enc_ref, w_ref,
                  cd_ref, ew_ref):
    h = enc_ref.shape[2]
    eq = (ids_col_ref[0] == ids_row_ref[0]).astype(jnp.bfloat16)
    cd_ref[0] = jnp.dot(caw_bf_ref[0], eq,
                        preferred_element_type=jnp.float32)
    ew_ref[0] = jnp.sum(enc_ref[0] * w_ref[0][:h][None, :], axis=1,
                        keepdims=True)


def _dense_body(caw_ref, dec_ref, ew_ref, x_ref, w_ref, b_ref,
                out_ref, m_ref, sc_ref, og_ref):
    tt = caw_ref.shape[1]
    h = dec_ref.shape[2]
    v = x_ref.shape[2]

    w = w_ref[0]
    ew = ew_ref[0][:, 0]
    g = (jnp.sum(caw_ref[0] * ew[None, :], axis=1, keepdims=True)
         + jnp.sum(dec_ref[0] * w[h:][None, :], axis=1, keepdims=True)
         + b_ref[0, 0])
    pg = jax.nn.sigmoid(g)

    starts = list(range(0, v, VT))
    widths = [min(VT, v - s0) for s0 in starts]

    m = jnp.full((tt, 1), -jnp.inf, jnp.float32)
    for s0, w_ in zip(starts, widths):
        m = jnp.maximum(m, jnp.max(x_ref[0, :, s0:s0 + w_], axis=1,
                                   keepdims=True))

    z = jnp.zeros((tt, 1), jnp.float32)
    for s0, w_ in zip(starts, widths):
        e = jnp.exp(x_ref[0, :, s0:s0 + w_] - m)
        out_ref[0, :, s0:s0 + w_] = e
        z = z + jnp.sum(e, axis=1, keepdims=True)

    scale = pg / z
    for s0, w_ in zip(starts, widths):
        e = out_ref[0, :, s0:s0 + w_]
        out_ref[0, :, s0:s0 + w_] = jnp.log(scale * e + 1e-12)

    m_ref[0] = m
    sc_ref[0] = scale
    og_ref[0] = 1.0 - pg


def _fix_body(xg_ref, cd_ref, m_ref, sc_ref, og_ref, vals_ref):
    e = jnp.exp(xg_ref[0] - m_ref[0])
    vals_ref[0] = jnp.log(sc_ref[0] * e + og_ref[0] * cd_ref[0] + 1e-12)


LAG = 64


def _gather_body(x_flat, idx_hbm, xg_hbm, idx_v, xg_v, sem):
    wid = lax.axis_index("s") * NC + lax.axis_index("c")
    pltpu.sync_copy(idx_hbm.at[wid], idx_v)
    k = idx_v.shape[0]

    def fire(j, c):
        pltpu.async_copy(x_flat.at[idx_v.at[j]], xg_v.at[j], sem)

        @pl.when(j >= LAG)
        def _drain_one():
            pltpu.make_async_copy(xg_hbm.at[wid].at[j], xg_v.at[j],
                                  sem).wait()

        return c

    lax.fori_loop(0, k, fire, 0)
    pltpu.make_async_copy(xg_hbm.at[wid].at[pl.ds(0, LAG)],
                          xg_v.at[pl.ds(0, LAG)], sem).wait()
    pltpu.sync_copy(xg_v, xg_hbm.at[wid])


def _patch_body(base_hbm, vals_hbm, ids_hbm, out_hbm, ids_v, vals_v,
                rb, sem_i0, sem_i1, sem_o0, sem_o1):
    b_count, t_tgt, v = base_hbm.shape
    s_src = ids_hbm.shape[1]
    tpb = NW // b_count
    rows = t_tgt // tpb
    pairs = rows // 2
    wid = lax.axis_index("s") * NC + lax.axis_index("c")
    bi = wid // tpb
    t0 = (wid % tpb) * rows
    pltpu.sync_copy(ids_hbm.at[bi], ids_v)
    pltpu.sync_copy(vals_hbm.at[bi, pl.ds(t0, rows), :], vals_v)
    nchunks = s_src // 16
    sem_i = (sem_i0, sem_i1)
    sem_o = (sem_o0, sem_o1)

    def patch_row(r, slot):
        def patch(j, c2):
            vidx = ids_v[pl.ds(j * 16, 16)]
            vval = vals_v[r, pl.ds(j * 16, 16)]
            plsc.store_scatter(rb.at[slot], [vidx], vval)
            return c2

        lax.fori_loop(0, nchunks, patch, 0)

    def drain(slot, sem):
        pltpu.make_async_copy(base_hbm.at[bi, t0, :], rb.at[slot],
                              sem).wait()

    pltpu.async_copy(base_hbm.at[bi, t0, :], rb.at[0], sem_i0)
    pltpu.async_copy(base_hbm.at[bi, t0 + 1, :], rb.at[1], sem_i1)
    for rp in range(pairs):
        for sl in range(2):
            r = 2 * rp + sl
            drain(sl, sem_i[sl])
            patch_row(r, sl)
            pltpu.async_copy(rb.at[sl], out_hbm.at[bi, t0 + r, :],
                             sem_o[sl])
        if rp + 1 < pairs:
            for sl in range(2):
                drain(sl, sem_o[sl])
                pltpu.async_copy(base_hbm.at[bi, t0 + 2 * rp + 2 + sl, :],
                                 rb.at[sl], sem_i[sl])
    drain(0, sem_o0)
    drain(1, sem_o1)


def kernel(decoder_hidden_states, cross_attention_weights,
           encoder_hidden_states, vocab_logits, source_ids, vocab_size,
           W_gen, b_gen):
    b, t_tgt, h = decoder_hidden_states.shape
    t_src = encoder_hidden_states.shape[1]
    v = vocab_logits.shape[2]
    n_idx = b * t_tgt * t_src
    k = n_idx // (NW * CH)

    caw = cross_attention_weights
    caw_bf = caw.astype(jnp.bfloat16)
    ids = source_ids.astype(jnp.int32)
    ids_col = ids.reshape(b, t_src, 1)
    ids_row = ids.reshape(b, 1, t_src)
    w2 = W_gen.reshape(1, 2 * h)
    b2 = b_gen.reshape(1, 1)

    cd, ew = pl.pallas_call(
        _compact_body,
        grid=(b,),
        in_specs=[
            pl.BlockSpec((1, t_tgt, t_src), lambda i: (i, 0, 0)),
            pl.BlockSpec((1, t_src, 1), lambda i: (i, 0, 0)),
            pl.BlockSpec((1, 1, t_src), lambda i: (i, 0, 0)),
            pl.BlockSpec((1, t_src, h), lambda i: (i, 0, 0)),
            pl.BlockSpec((1, 2 * h), lambda i: (0, 0)),
        ],
        out_specs=[
            pl.BlockSpec((1, t_tgt, t_src), lambda i: (i, 0, 0)),
            pl.BlockSpec((1, t_src, 1), lambda i: (i, 0, 0)),
        ],
        out_shape=[
            jax.ShapeDtypeStruct((b, t_tgt, t_src), jnp.float32),
            jax.ShapeDtypeStruct((b, t_src, 1), jnp.float32),
        ],
    )(caw_bf, ids_col, ids_row, encoder_hidden_states, w2)

    fidx = ((jnp.arange(b * t_tgt, dtype=jnp.int32) * v)
            .reshape(b, t_tgt, 1) + ids[:, None, :])

    nt = t_tgt // TT
    base, m, sc, og = pl.pallas_call(
        _dense_body,
        grid=(b, nt),
        in_specs=[
            pl.BlockSpec((1, TT, t_src), lambda i, t: (i, t, 0)),
            pl.BlockSpec((1, TT, h), lambda i, t: (i, t, 0)),
            pl.BlockSpec((1, t_src, 1), lambda i, t: (i, 0, 0)),
            pl.BlockSpec((1, TT, v), lambda i, t: (i, t, 0)),
            pl.BlockSpec((1, 2 * h), lambda i, t: (0, 0)),
            pl.BlockSpec((1, 1), lambda i, t: (0, 0)),
        ],
        out_specs=[
            pl.BlockSpec((1, TT, v), lambda i, t: (i, t, 0)),
            pl.BlockSpec((1, TT, 1), lambda i, t: (i, t, 0)),
            pl.BlockSpec((1, TT, 1), lambda i, t: (i, t, 0)),
            pl.BlockSpec((1, TT, 1), lambda i, t: (i, t, 0)),
        ],
        out_shape=[
            jax.ShapeDtypeStruct((b, t_tgt, v), jnp.float32),
            jax.ShapeDtypeStruct((b, t_tgt, 1), jnp.float32),
            jax.ShapeDtypeStruct((b, t_tgt, 1), jnp.float32),
            jax.ShapeDtypeStruct((b, t_tgt, 1), jnp.float32),
        ],
    )(caw, decoder_hidden_states, ew, vocab_logits, w2, b2)

    mesh = plsc.VectorSubcoreMesh(core_axis_name="c", subcore_axis_name="s",
                                  num_cores=NC, num_subcores=NS)
    idx3 = fidx.reshape(NW, k, CH)
    x_flat = vocab_logits.reshape(-1)
    gather_k = pl.kernel(
        _gather_body,
        out_type=jax.ShapeDtypeStruct((NW, k, CH), jnp.float32),
        mesh=mesh,
        scratch_types=[
            pltpu.VMEM((k, CH), jnp.int32),
            pltpu.VMEM((k, CH), jnp.float32),
            pltpu.SemaphoreType.DMA,
        ],
    )
    xg = gather_k(x_flat, idx3)

    vals = pl.pallas_call(
        _fix_body,
        grid=(b,),
        in_specs=[
            pl.BlockSpec((1, t_tgt, t_src), lambda i: (i, 0, 0)),
            pl.BlockSpec((1, t_tgt, t_src), lambda i: (i, 0, 0)),
            pl.BlockSpec((1, t_tgt, 1), lambda i: (i, 0, 0)),
            pl.BlockSpec((1, t_tgt, 1), lambda i: (i, 0, 0)),
            pl.BlockSpec((1, t_tgt, 1), lambda i: (i, 0, 0)),
        ],
        out_specs=pl.BlockSpec((1, t_tgt, t_src), lambda i: (i, 0, 0)),
        out_shape=jax.ShapeDtypeStruct((b, t_tgt, t_src), jnp.float32),
    )(xg.reshape(b, t_tgt, t_src), cd, m, sc, og)

    rows = t_tgt // (NW // b)
    patch_k = pl.kernel(
        _patch_body,
        out_type=jax.ShapeDtypeStruct((b, t_tgt, v), jnp.float32),
        mesh=mesh,
        compiler_params=pltpu.CompilerParams(use_tc_tiling_on_sc=False, needs_layout_passes=False),
        scratch_types=[
            pltpu.VMEM((t_src,), jnp.int32),
            pltpu.VMEM((rows, t_src), jnp.float32),
            pltpu.VMEM((2, v), jnp.float32),
            pltpu.SemaphoreType.DMA,
            pltpu.SemaphoreType.DMA,
            pltpu.SemaphoreType.DMA,
            pltpu.SemaphoreType.DMA,
        ],
    )
    return patch_k(base, vals, ids)


# speedup vs baseline: 1.0679x; 1.0002x over previous
import functools

import jax
import jax.numpy as jnp
from jax import lax
from jax.experimental import pallas as pl
from jax.experimental.pallas import tpu as pltpu
from jax.experimental.pallas import tpu_sc as plsc

TT = 64
VT = 2048
NC, NS = 2, 16
NW = NC * NS
CH = 128


def _compact_body(caw_bf_ref, ids_col_ref, ids_row_ref, enc_ref, w_ref,
                  cd_ref, ew_ref):
    h = enc_ref.shape[2]
    eq = (ids_col_ref[0] == ids_row_ref[0]).astype(jnp.bfloat16)
    cd_ref[0] = jnp.dot(caw_bf_ref[0], eq,
                        preferred_element_type=jnp.float32)
    ew_ref[0] = jnp.sum(enc_ref[0] * w_ref[0][:h][None, :], axis=1,
                        keepdims=True)


def _dense_body(caw_ref, dec_ref, ew_ref, x_ref, w_ref, b_ref,
                out_ref, m_ref, sc_ref, og_ref):
    tt = caw_ref.shape[1]
    h = dec_ref.shape[2]
    v = x_ref.shape[2]

    w = w_ref[0]
    ew = ew_ref[0][:, 0]
    g = (jnp.sum(caw_ref[0] * ew[None, :], axis=1, keepdims=True)
         + jnp.sum(dec_ref[0] * w[h:][None, :], axis=1, keepdims=True)
         + b_ref[0, 0])
    pg = jax.nn.sigmoid(g)

    starts = list(range(0, v, VT))
    widths = [min(VT, v - s0) for s0 in starts]

    m = jnp.full((tt, 1), -jnp.inf, jnp.float32)
    for s0, w_ in zip(starts, widths):
        m = jnp.maximum(m, jnp.max(x_ref[0, :, s0:s0 + w_], axis=1,
                                   keepdims=True))

    z = jnp.zeros((tt, 1), jnp.float32)
    for s0, w_ in zip(starts, widths):
        e = jnp.exp(x_ref[0, :, s0:s0 + w_] - m)
        out_ref[0, :, s0:s0 + w_] = e
        z = z + jnp.sum(e, axis=1, keepdims=True)

    scale = pg / z
    for s0, w_ in zip(starts, widths):
        e = out_ref[0, :, s0:s0 + w_]
        out_ref[0, :, s0:s0 + w_] = jnp.log(scale * e + 1e-12)

    m_ref[0] = m
    sc_ref[0] = scale
    og_ref[0] = 1.0 - pg


def _fix_body(xg_ref, cd_ref, m_ref, sc_ref, og_ref, vals_ref):
    e = jnp.exp(xg_ref[0] - m_ref[0])
    vals_ref[0] = jnp.log(sc_ref[0] * e + og_ref[0] * cd_ref[0] + 1e-12)


LAG = 64


def _gather_body(x_flat, idx_hbm, xg_hbm, idx_v, xg_v, sem):
    wid = lax.axis_index("s") * NC + lax.axis_index("c")
    pltpu.sync_copy(idx_hbm.at[wid], idx_v)
    k = idx_v.shape[0]

    def fire(j, c):
        pltpu.async_copy(x_flat.at[idx_v.at[j]], xg_v.at[j], sem)

        @pl.when(j >= LAG)
        def _drain_one():
            pltpu.make_async_copy(xg_hbm.at[wid].at[j], xg_v.at[j],
                                  sem).wait()

        return c

    lax.fori_loop(0, k, fire, 0)
    pltpu.make_async_copy(xg_hbm.at[wid].at[pl.ds(0, LAG)],
                          xg_v.at[pl.ds(0, LAG)], sem).wait()
    pltpu.sync_copy(xg_v, xg_hbm.at[wid])


def _patch_body(base_hbm, vals_hbm, ids_hbm, out_hbm, ids_v, vals_v,
                rb, sem_i0, sem_i1, sem_o0, sem_o1):
    b_count, t_tgt, v = base_hbm.shape
    s_src = ids_hbm.shape[1]
    tpb = NW // b_count
    rows = t_tgt // tpb
    pairs = rows // 2
    wid = lax.axis_index("s") * NC + lax.axis_index("c")
    bi = wid // tpb
    t0 = (wid % tpb) * rows
    pltpu.sync_copy(ids_hbm.at[bi], ids_v)
    pltpu.sync_copy(vals_hbm.at[bi, pl.ds(t0, rows), :], vals_v)
    nchunks = s_src // 16
    sem_i = (sem_i0, sem_i1)
    sem_o = (sem_o0, sem_o1)

    def patch_row(r, slot):
        def patch(j, c2):
            vidx = ids_v[pl.ds(j * 16, 16)]
            vval = vals_v[r, pl.ds(j * 16, 16)]
            plsc.store_scatter(rb.at[slot], [vidx], vval)
            return c2

        lax.fori_loop(0, nchunks, patch, 0)

    def drain(slot, sem):
        pltpu.make_async_copy(base_hbm.at[bi, t0, :], rb.at[slot],
                              sem).wait()

    pltpu.async_copy(base_hbm.at[bi, t0, :], rb.at[0], sem_i0)
    pltpu.async_copy(base_hbm.at[bi, t0 + 1, :], rb.at[1], sem_i1)
    for rp in range(pairs):
        for sl in range(2):
            r = 2 * rp + sl
            drain(sl, sem_i[sl])
            patch_row(r, sl)
            pltpu.async_copy(rb.at[sl], out_hbm.at[bi, t0 + r, :],
                             sem_o[sl])
        if rp + 1 < pairs:
            for sl in range(2):
                drain(sl, sem_o[sl])
                pltpu.async_copy(base_hbm.at[bi, t0 + 2 * rp + 2 + sl, :],
                                 rb.at[sl], sem_i[sl])
    drain(0, sem_o0)
    drain(1, sem_o1)


def kernel(decoder_hidden_states, cross_attention_weights,
           encoder_hidden_states, vocab_logits, source_ids, vocab_size,
           W_gen, b_gen):
    b, t_tgt, h = decoder_hidden_states.shape
    t_src = encoder_hidden_states.shape[1]
    v = vocab_logits.shape[2]
    n_idx = b * t_tgt * t_src
    k = n_idx // (NW * CH)

    caw = cross_attention_weights
    caw_bf = caw.astype(jnp.bfloat16)
    ids = source_ids.astype(jnp.int32)
    ids_col = ids.reshape(b, t_src, 1)
    ids_row = ids.reshape(b, 1, t_src)
    w2 = W_gen.reshape(1, 2 * h)
    b2 = b_gen.reshape(1, 1)

    cd, ew = pl.pallas_call(
        _compact_body,
        grid=(b,),
        in_specs=[
            pl.BlockSpec((1, t_tgt, t_src), lambda i: (i, 0, 0)),
            pl.BlockSpec((1, t_src, 1), lambda i: (i, 0, 0)),
            pl.BlockSpec((1, 1, t_src), lambda i: (i, 0, 0)),
            pl.BlockSpec((1, t_src, h), lambda i: (i, 0, 0)),
            pl.BlockSpec((1, 2 * h), lambda i: (0, 0)),
        ],
        out_specs=[
            pl.BlockSpec((1, t_tgt, t_src), lambda i: (i, 0, 0)),
            pl.BlockSpec((1, t_src, 1), lambda i: (i, 0, 0)),
        ],
        out_shape=[
            jax.ShapeDtypeStruct((b, t_tgt, t_src), jnp.float32),
            jax.ShapeDtypeStruct((b, t_src, 1), jnp.float32),
        ],
    )(caw_bf, ids_col, ids_row, encoder_hidden_states, w2)

    fidx = ((jnp.arange(b * t_tgt, dtype=jnp.int32) * v)
            .reshape(b, t_tgt, 1) + ids[:, None, :])

    nt = t_tgt // TT
    base, m, sc, og = pl.pallas_call(
        _dense_body,
        grid=(b, nt),
        in_specs=[
            pl.BlockSpec((1, TT, t_src), lambda i, t: (i, t, 0)),
            pl.BlockSpec((1, TT, h), lambda i, t: (i, t, 0)),
            pl.BlockSpec((1, t_src, 1), lambda i, t: (i, 0, 0)),
            pl.BlockSpec((1, TT, v), lambda i, t: (i, t, 0)),
            pl.BlockSpec((1, 2 * h), lambda i, t: (0, 0)),
            pl.BlockSpec((1, 1), lambda i, t: (0, 0)),
        ],
        out_specs=[
            pl.BlockSpec((1, TT, v), lambda i, t: (i, t, 0)),
            pl.BlockSpec((1, TT, 1), lambda i, t: (i, t, 0)),
            pl.BlockSpec((1, TT, 1), lambda i, t: (i, t, 0)),
            pl.BlockSpec((1, TT, 1), lambda i, t: (i, t, 0)),
        ],
        out_shape=[
            jax.ShapeDtypeStruct((b, t_tgt, v), jnp.float32),
            jax.ShapeDtypeStruct((b, t_tgt, 1), jnp.float32),
            jax.ShapeDtypeStruct((b, t_tgt, 1), jnp.float32),
            jax.ShapeDtypeStruct((b, t_tgt, 1), jnp.float32),
        ],
    )(caw, decoder_hidden_states, ew, vocab_logits, w2, b2)

    mesh = plsc.VectorSubcoreMesh(core_axis_name="c", subcore_axis_name="s",
                                  num_cores=NC, num_subcores=NS)
    idx3 = fidx.reshape(NW, k, CH)
    x_flat = vocab_logits.reshape(-1)
    gather_k = pl.kernel(
        _gather_body,
        out_type=jax.ShapeDtypeStruct((NW, k, CH), jnp.float32),
        mesh=mesh,
        scratch_types=[
            pltpu.VMEM((k, CH), jnp.int32),
            pltpu.VMEM((k, CH), jnp.float32),
            pltpu.SemaphoreType.DMA,
        ],
    )
    xg = gather_k(x_flat, idx3)

    vals = pl.pallas_call(
        _fix_body,
        grid=(b,),
        in_specs=[
            pl.BlockSpec((1, t_tgt, t_src), lambda i: (i, 0, 0)),
            pl.BlockSpec((1, t_tgt, t_src), lambda i: (i, 0, 0)),
            pl.BlockSpec((1, t_tgt, 1), lambda i: (i, 0, 0)),
            pl.BlockSpec((1, t_tgt, 1), lambda i: (i, 0, 0)),
            pl.BlockSpec((1, t_tgt, 1), lambda i: (i, 0, 0)),
        ],
        out_specs=pl.BlockSpec((1, t_tgt, t_src), lambda i: (i, 0, 0)),
        out_shape=jax.ShapeDtypeStruct((b, t_tgt, t_src), jnp.float32),
    )(xg.reshape(b, t_tgt, t_src), cd, m, sc, og)

    rows = t_tgt // (NW // b)
    patch_k = pl.kernel(
        _patch_body,
        out_type=jax.ShapeDtypeStruct((b, t_tgt, v), jnp.float32),
        mesh=mesh,
        compiler_params=pltpu.CompilerParams(use_tc_tiling_on_sc=False, needs_layout_passes=False),
        scratch_types=[
            pltpu.VMEM((t_src,), jnp.int32),
            pltpu.VMEM((rows, t_src), jnp.float32),
            pltpu.VMEM((2, v), jnp.float32),
            pltpu.SemaphoreType.DMA,
            pltpu.SemaphoreType.DMA,
            pltpu.SemaphoreType.DMA,
            pltpu.SemaphoreType.DMA,
        ],
    )
    return patch_k(base, vals, ids)
